# baseline (speedup 1.0000x reference)
import functools

import jax
import jax.numpy as jnp
from jax import lax
from jax.experimental import pallas as pl
from jax.experimental.pallas import tpu as pltpu
from jax.experimental.pallas import tpu_sc as plsc

N = 10000
C = 128
H = 64
E = 320000
NC = 2
NS = 16
G = 80
CHUNKS = E // NS // G
TILE_ROWS = 624
TAIL_ROWS = N - NS * TILE_ROWS
LANES = 16


def _bn_body(x_ref, gamma_ref, beta_ref, h_ref):
    xv = x_ref[...]
    mean = jnp.mean(xv, axis=0, keepdims=True)
    xc = xv - mean
    var = jnp.mean(xc * xc, axis=0, keepdims=True)
    hv = xc * lax.rsqrt(var + 1e-5) * gamma_ref[...] + beta_ref[...]
    h_ref[0] = hv[:, :H]
    h_ref[1] = hv[:, H:]


def _bn(x, gamma, beta):
    return pl.pallas_call(
        _bn_body,
        out_shape=jax.ShapeDtypeStruct((NC, N, H), jnp.float32),
    )(x, gamma, beta)


def _scale_chunk(st_w, rows_v, slot, b):
    for g in range(G // LANES):
        wrow = st_w[slot, b, pl.ds(g * LANES, LANES)]
        for l in range(LANES):
            e = g * LANES + l
            for c4 in range(H // LANES):
                sl = pl.ds(c4 * LANES, LANES)
                rows_v[e, sl] = rows_v[e, sl] * wrow[l]


NBUF = 5
ROUNDS = CHUNKS // NBUF
NSLOT = 3


def _spmm_body(h_ref, src_ref, dst_ref, w_ref, zero_ref, out_ref,
               st_src, st_dst, st_w, bufs, h_sh, acc,
               hsem, stsems, gsems, ssems):
    cid = lax.axis_index("c")
    sid = lax.axis_index("s")
    base = sid * TILE_ROWS
    tailb = NS * TILE_ROWS

    def stage(r, slot):
        return (pltpu.make_async_copy(src_ref.at[sid, r], st_src.at[slot],
                                      stsems.at[slot]),
                pltpu.make_async_copy(dst_ref.at[sid, r], st_dst.at[slot],
                                      stsems.at[slot]),
                pltpu.make_async_copy(w_ref.at[sid, r], st_w.at[slot],
                                      stsems.at[slot]))

    def stage_start(r, slot):
        for d in stage(r, slot):
            d.start()

    def stage_wait(r, slot):
        for d in stage(r, slot):
            d.wait()

    def gather(b, slot):
        return pltpu.make_async_copy(
            h_sh.at[st_src.at[slot].at[b]], bufs.at[b], gsems.at[b])

    def scatter(b, slot):
        return pltpu.make_async_copy(
            bufs.at[b], acc.at[st_dst.at[slot].at[b]], ssems.at[b])

    stage_start(0, 0)
    stage_start(1, 1)
    for cc in range(NC):
        @pl.when(cid == cc)
        def _(cc=cc):
            hcp = pltpu.make_async_copy(
                h_ref.at[cc].at[pl.ds(base, TILE_ROWS)],
                h_sh.at[pl.ds(base, TILE_ROWS)], hsem)
            hcp.start()
            pltpu.sync_copy(zero_ref, acc.at[pl.ds(base, TILE_ROWS)])

            @pl.when(sid == NS - 1)
            def _():
                pltpu.sync_copy(h_ref.at[cc].at[pl.ds(tailb, TAIL_ROWS)],
                                h_sh.at[pl.ds(tailb, TAIL_ROWS)])
                pltpu.sync_copy(zero_ref.at[pl.ds(0, TAIL_ROWS)],
                                acc.at[pl.ds(tailb, TAIL_ROWS)])

            hcp.wait()

    plsc.subcore_barrier()

    stage_wait(0, 0)
    for b in range(NBUF - 1):
        gather(b, 0).start()

    def round_(r, carry):
        slot = lax.rem(r, NSLOT)
        slot_next = lax.rem(r + 1, NSLOT)
        slot_prev = lax.rem(r + 2, NSLOT)
        for b in range(NBUF):
            gather(b, slot).wait()
            _scale_chunk(st_w, bufs.at[b], slot, b)
            scatter(b, slot).start(add=True)
            if b == 0:
                @pl.when(r > 0)
                def _():
                    scatter(NBUF - 1, slot_prev).wait()

                @pl.when(r + 2 < ROUNDS)
                def _():
                    stage_start(r + 2, slot_prev)

                gather(NBUF - 1, slot).start()
            else:
                scatter(b - 1, slot).wait()

                @pl.when(r + 1 < ROUNDS)
                def _(b=b):
                    if b == 1:
                        stage_wait(r + 1, slot_next)
                    gather(b - 1, slot_next).start()
        return carry

    lax.fori_loop(0, ROUNDS, round_, 0)
    scatter(NBUF - 1, (ROUNDS - 1) % NSLOT).wait()

    plsc.subcore_barrier()
    for cc in range(NC):
        @pl.when(cid == cc)
        def _(cc=cc):
            pltpu.sync_copy(acc.at[pl.ds(base, TILE_ROWS)],
                            out_ref.at[cc].at[pl.ds(base, TILE_ROWS)])

            @pl.when(sid == NS - 1)
            def _():
                pltpu.sync_copy(acc.at[pl.ds(tailb, TAIL_ROWS)],
                                out_ref.at[cc].at[pl.ds(tailb, TAIL_ROWS)])


@functools.cache
def _make_spmm():
    return pl.kernel(
        _spmm_body,
        out_type=jax.ShapeDtypeStruct((NC, N, H), jnp.float32),
        mesh=plsc.VectorSubcoreMesh(core_axis_name="c", subcore_axis_name="s",
                                    num_cores=NC, num_subcores=NS),
        compiler_params=pltpu.CompilerParams(use_tc_tiling_on_sc=False),
        scratch_types=[
            pltpu.VMEM((NSLOT, NBUF, G), jnp.int32),
            pltpu.VMEM((NSLOT, NBUF, G), jnp.int32),
            pltpu.VMEM((NSLOT, NBUF, G), jnp.float32),
            pltpu.VMEM((NBUF, G, H), jnp.float32),
            pltpu.VMEM_SHARED((N, H), jnp.float32),
            pltpu.VMEM_SHARED((N, H), jnp.float32),
            pltpu.SemaphoreType.DMA,
            pltpu.SemaphoreType.DMA((NSLOT,)),
            pltpu.SemaphoreType.DMA((NBUF,)),
            pltpu.SemaphoreType.DMA((NBUF,)),
        ],
    )


def _mm_body(h_ref, t1_ref, u_ref, wa_ref, w1_ref, wc_ref, b_ref, o_ref):
    hv = jnp.concatenate([h_ref[0], h_ref[1]], axis=1)
    t1v = jnp.concatenate([t1_ref[0], t1_ref[1]], axis=1)
    uv = jnp.concatenate([u_ref[0], u_ref[1]], axis=1)
    acc = jnp.dot(hv, wa_ref[...], preferred_element_type=jnp.float32)
    acc = acc + jnp.dot(t1v, w1_ref[...], preferred_element_type=jnp.float32)
    acc = acc + jnp.dot(uv, wc_ref[...], preferred_element_type=jnp.float32)
    acc = acc + b_ref[...]
    o_ref[...] = jnp.maximum(acc, 0.0)


def _mm(h, t1, u, wa, w1, wc, b):
    return pl.pallas_call(
        _mm_body,
        out_shape=jax.ShapeDtypeStruct((N, C), jnp.float32),
    )(h, t1, u, wa, w1, wc, b)


def kernel(x, edge_index, edge_weight, W, b, gamma, beta):
    src = edge_index[0].reshape(NS, ROUNDS, NBUF, G)
    dst = edge_index[1].reshape(NS, ROUNDS, NBUF, G)
    w3 = edge_weight.reshape(NS, ROUNDS, NBUF, G)
    zeros = jnp.zeros((TILE_ROWS, H), jnp.float32)
    h = _bn(x, gamma.reshape(1, C), beta.reshape(1, C))
    spmm = _make_spmm()
    t1 = spmm(h, src, dst, w3, zeros)
    u = spmm(t1, src, dst, w3, zeros)
    wa = W[0] - W[2]
    wc = 2.0 * W[2]
    return _mm(h, t1, u, wa, W[1], wc, b.reshape(1, C))


# speedup vs baseline: 1.0176x; 1.0176x over previous
import functools

import jax
import jax.numpy as jnp
from jax import lax
from jax.experimental import pallas as pl
from jax.experimental.pallas import tpu as pltpu
from jax.experimental.pallas import tpu_sc as plsc

N = 10000
C = 128
H = 64
E = 320000
NC = 2
NS = 16
G = 80
CHUNKS = E // NS // G
TILE_ROWS = 624
TAIL_ROWS = N - NS * TILE_ROWS
LANES = 16


def _bn_body(x_ref, gamma_ref, beta_ref, h_ref):
    xv = x_ref[...]
    mean = jnp.mean(xv, axis=0, keepdims=True)
    xc = xv - mean
    var = jnp.mean(xc * xc, axis=0, keepdims=True)
    hv = xc * lax.rsqrt(var + 1e-5) * gamma_ref[...] + beta_ref[...]
    h_ref[0] = hv[:, :H]
    h_ref[1] = hv[:, H:]


def _bn(x, gamma, beta):
    return pl.pallas_call(
        _bn_body,
        out_shape=jax.ShapeDtypeStruct((NC, N, H), jnp.float32),
    )(x, gamma, beta)


def _scale_chunk(st_w, rows_v, slot, b):
    for g in range(G // LANES):
        wrow = st_w[slot, b, pl.ds(g * LANES, LANES)]
        for l in range(LANES):
            e = g * LANES + l
            for c4 in range(H // LANES):
                sl = pl.ds(c4 * LANES, LANES)
                rows_v[e, sl] = rows_v[e, sl] * wrow[l]


NBUF = 5
ROUNDS = CHUNKS // NBUF
NSLOT = 3


def _ring(src_parent, acc, sid, src_ref, dst_ref, w_ref,
          st_src, st_dst, st_w, bufs, stsems, gsems, ssems):
    def stage(r, slot):
        return (pltpu.make_async_copy(src_ref.at[sid, r], st_src.at[slot],
                                      stsems.at[slot]),
                pltpu.make_async_copy(dst_ref.at[sid, r], st_dst.at[slot],
                                      stsems.at[slot]),
                pltpu.make_async_copy(w_ref.at[sid, r], st_w.at[slot],
                                      stsems.at[slot]))

    def stage_start(r, slot):
        for d in stage(r, slot):
            d.start()

    def stage_wait(r, slot):
        for d in stage(r, slot):
            d.wait()

    def gather(b, slot):
        return pltpu.make_async_copy(
            src_parent.at[st_src.at[slot].at[b]], bufs.at[b], gsems.at[b])

    def scatter(b, slot):
        return pltpu.make_async_copy(
            bufs.at[b], acc.at[st_dst.at[slot].at[b]], ssems.at[b])

    stage_start(0, 0)
    stage_start(1, 1)
    stage_wait(0, 0)
    for b in range(NBUF - 1):
        gather(b, 0).start()

    def round_(r, carry):
        slot = lax.rem(r, NSLOT)
        slot_next = lax.rem(r + 1, NSLOT)
        slot_prev = lax.rem(r + 2, NSLOT)
        for b in range(NBUF):
            gather(b, slot).wait()
            _scale_chunk(st_w, bufs.at[b], slot, b)
            scatter(b, slot).start(add=True)
            if b == 0:
                @pl.when(r > 0)
                def _():
                    scatter(NBUF - 1, slot_prev).wait()

                @pl.when(r + 2 < ROUNDS)
                def _():
                    stage_start(r + 2, slot_prev)

                gather(NBUF - 1, slot).start()
            else:
                scatter(b - 1, slot).wait()

                @pl.when(r + 1 < ROUNDS)
                def _(b=b):
                    if b == 1:
                        stage_wait(r + 1, slot_next)
                    gather(b - 1, slot_next).start()
        return carry

    lax.fori_loop(0, ROUNDS, round_, 0)
    scatter(NBUF - 1, (ROUNDS - 1) % NSLOT).wait()


def _cheb_body(h_ref, src_ref, dst_ref, w_ref, zero_ref, t1_ref, u_ref,
               st_src, st_dst, st_w, bufs, h_sh, acc,
               hsem, osem, stsems, gsems, ssems):
    cid = lax.axis_index("c")
    sid = lax.axis_index("s")
    base = sid * TILE_ROWS
    tailb = NS * TILE_ROWS
    ring_args = (sid, src_ref, dst_ref, w_ref,
                 st_src, st_dst, st_w, bufs, stsems, gsems, ssems)

    for cc in range(NC):
        @pl.when(cid == cc)
        def _(cc=cc):
            hcp = pltpu.make_async_copy(
                h_ref.at[cc].at[pl.ds(base, TILE_ROWS)],
                h_sh.at[pl.ds(base, TILE_ROWS)], hsem)
            hcp.start()
            pltpu.sync_copy(zero_ref, acc.at[pl.ds(base, TILE_ROWS)])

            @pl.when(sid == NS - 1)
            def _():
                pltpu.sync_copy(h_ref.at[cc].at[pl.ds(tailb, TAIL_ROWS)],
                                h_sh.at[pl.ds(tailb, TAIL_ROWS)])
                pltpu.sync_copy(zero_ref.at[pl.ds(0, TAIL_ROWS)],
                                acc.at[pl.ds(tailb, TAIL_ROWS)])

            hcp.wait()

    plsc.subcore_barrier()
    _ring(h_sh, acc, *ring_args)
    plsc.subcore_barrier()

    for cc in range(NC):
        @pl.when(cid == cc)
        def _(cc=cc):
            o1 = pltpu.make_async_copy(
                acc.at[pl.ds(base, TILE_ROWS)],
                t1_ref.at[cc].at[pl.ds(base, TILE_ROWS)], osem)
            o1.start()
            pltpu.sync_copy(zero_ref, h_sh.at[pl.ds(base, TILE_ROWS)])

            @pl.when(sid == NS - 1)
            def _():
                pltpu.sync_copy(acc.at[pl.ds(tailb, TAIL_ROWS)],
                                t1_ref.at[cc].at[pl.ds(tailb, TAIL_ROWS)])
                pltpu.sync_copy(zero_ref.at[pl.ds(0, TAIL_ROWS)],
                                h_sh.at[pl.ds(tailb, TAIL_ROWS)])

            o1.wait()

    plsc.subcore_barrier()
    _ring(acc, h_sh, *ring_args)
    plsc.subcore_barrier()

    for cc in range(NC):
        @pl.when(cid == cc)
        def _(cc=cc):
            pltpu.sync_copy(h_sh.at[pl.ds(base, TILE_ROWS)],
                            u_ref.at[cc].at[pl.ds(base, TILE_ROWS)])

            @pl.when(sid == NS - 1)
            def _():
                pltpu.sync_copy(h_sh.at[pl.ds(tailb, TAIL_ROWS)],
                                u_ref.at[cc].at[pl.ds(tailb, TAIL_ROWS)])


@functools.cache
def _make_cheb():
    return pl.kernel(
        _cheb_body,
        out_type=(jax.ShapeDtypeStruct((NC, N, H), jnp.float32),
                  jax.ShapeDtypeStruct((NC, N, H), jnp.float32)),
        mesh=plsc.VectorSubcoreMesh(core_axis_name="c", subcore_axis_name="s",
                                    num_cores=NC, num_subcores=NS),
        compiler_params=pltpu.CompilerParams(use_tc_tiling_on_sc=False),
        scratch_types=[
            pltpu.VMEM((NSLOT, NBUF, G), jnp.int32),
            pltpu.VMEM((NSLOT, NBUF, G), jnp.int32),
            pltpu.VMEM((NSLOT, NBUF, G), jnp.float32),
            pltpu.VMEM((NBUF, G, H), jnp.float32),
            pltpu.VMEM_SHARED((N, H), jnp.float32),
            pltpu.VMEM_SHARED((N, H), jnp.float32),
            pltpu.SemaphoreType.DMA,
            pltpu.SemaphoreType.DMA,
            pltpu.SemaphoreType.DMA((NSLOT,)),
            pltpu.SemaphoreType.DMA((NBUF,)),
            pltpu.SemaphoreType.DMA((NBUF,)),
        ],
    )


def _mm_body(h_ref, t1_ref, u_ref, wa_ref, w1_ref, wc_ref, b_ref, o_ref):
    hv = jnp.concatenate([h_ref[0], h_ref[1]], axis=1)
    t1v = jnp.concatenate([t1_ref[0], t1_ref[1]], axis=1)
    uv = jnp.concatenate([u_ref[0], u_ref[1]], axis=1)
    acc = jnp.dot(hv, wa_ref[...], preferred_element_type=jnp.float32)
    acc = acc + jnp.dot(t1v, w1_ref[...], preferred_element_type=jnp.float32)
    acc = acc + jnp.dot(uv, wc_ref[...], preferred_element_type=jnp.float32)
    acc = acc + b_ref[...]
    o_ref[...] = jnp.maximum(acc, 0.0)


def _mm(h, t1, u, wa, w1, wc, b):
    return pl.pallas_call(
        _mm_body,
        out_shape=jax.ShapeDtypeStruct((N, C), jnp.float32),
    )(h, t1, u, wa, w1, wc, b)


def kernel(x, edge_index, edge_weight, W, b, gamma, beta):
    src = edge_index[0].reshape(NS, ROUNDS, NBUF, G)
    dst = edge_index[1].reshape(NS, ROUNDS, NBUF, G)
    w3 = edge_weight.reshape(NS, ROUNDS, NBUF, G)
    zeros = jnp.zeros((TILE_ROWS, H), jnp.float32)
    h = _bn(x, gamma.reshape(1, C), beta.reshape(1, C))
    t1, u = _make_cheb()(h, src, dst, w3, zeros)
    wa = W[0] - W[2]
    wc = 2.0 * W[2]
    return _mm(h, t1, u, wa, W[1], wc, b.reshape(1, C))
